# Initial kernel scaffold; baseline (speedup 1.0000x reference)
import jax
import jax.numpy as jnp
from jax.experimental import pallas as pl


def kernel(x, edge_index, edge_weight, batch, W1, bn1_g, bn1_b, W2, bn2_g, bn2_b, lnW, lnb):
    raise NotImplementedError("write your pallas kernel here")


# speedup vs baseline: 14.8500x; 14.8500x over previous
import functools

import jax
import jax.numpy as jnp
from jax import lax
from jax.experimental import pallas as pl
from jax.experimental.pallas import tpu as pltpu
from jax.experimental.pallas import tpu_sc as plsc

N = 10000
F = 128
G = 64
NCLS = 4
EPS = 1e-5

NC = 2
NS = 16
NW = NC * NS
CHUNK = 128
CH = 79
PW = CH * CHUNK
EP = NW * PW
DPAD = 10240
DSEG = DPAD // NS
NROW = 10112
SEG = NROW // NS

_HI = jax.lax.Precision.HIGHEST

_MESH = plsc.VectorSubcoreMesh(
    core_axis_name="c", subcore_axis_name="s", num_cores=NC, num_subcores=NS)


def _deg_body(col_hbm, ew_hbm, out_hbm, col_v, ew_v, zbuf, deg_sh):
    c = lax.axis_index("c")
    s = lax.axis_index("s")
    wid = s * NC + c
    zv = jnp.zeros((16,), jnp.float32)

    def zero_zbuf(i, carry):
        zbuf[pl.ds(i * 16, 16)] = zv
        return carry
    lax.fori_loop(0, DSEG // 16, zero_zbuf, 0, unroll=True)
    pltpu.sync_copy(zbuf.at[pl.ds(0, DSEG)], deg_sh.at[pl.ds(s * DSEG, DSEG)])
    plsc.subcore_barrier()

    pltpu.sync_copy(col_hbm.at[wid], col_v)
    pltpu.sync_copy(ew_hbm.at[wid], ew_v)

    def chunk(j, carry):
        pltpu.sync_copy(ew_v.at[j], deg_sh.at[col_v.at[j]], add=True)
        return carry
    lax.fori_loop(0, CH, chunk, 0)
    plsc.subcore_barrier()
    pltpu.sync_copy(deg_sh.at[pl.ds(s * DSEG, DSEG)],
                    out_hbm.at[pl.ds(c * DPAD + s * DSEG, DSEG)])


_deg_kernel = functools.partial(
    pl.kernel,
    out_type=jax.ShapeDtypeStruct((NC * DPAD,), jnp.float32),
    mesh=_MESH,
    scratch_types=[
        pltpu.VMEM((CH, CHUNK), jnp.int32),
        pltpu.VMEM((CH, CHUNK), jnp.float32),
        pltpu.VMEM((DSEG,), jnp.float32),
        pltpu.VMEM_SHARED((DPAD,), jnp.float32),
    ],
)(_deg_body)


def _edge_body(hp_hbm, row_hbm, col_hbm, ew_hbm, out_hbm,
               row_v, col_v, ew_v, rows_v, acc_sh, sem):
    c = lax.axis_index("c")
    s = lax.axis_index("s")
    wid = s * NC + c
    zv = jnp.zeros((16,), jnp.float32)

    def zero_rows(i, carry):
        for f in range(F // 16):
            rows_v[i, pl.ds(f * 16, 16)] = zv
        return carry
    lax.fori_loop(0, CHUNK, zero_rows, 0)

    def zero_acc(i, carry):
        pltpu.sync_copy(rows_v.at[pl.ds(0, CHUNK)],
                        acc_sh.at[pl.ds(s * SEG + i * CHUNK, CHUNK)])
        return carry
    lax.fori_loop(0, 4, zero_acc, 0)
    pltpu.sync_copy(rows_v.at[pl.ds(0, SEG - 4 * CHUNK)],
                    acc_sh.at[pl.ds(s * SEG + 4 * CHUNK, SEG - 4 * CHUNK)])
    plsc.subcore_barrier()

    pltpu.sync_copy(row_hbm.at[wid], row_v)
    pltpu.sync_copy(col_hbm.at[wid], col_v)
    pltpu.sync_copy(ew_hbm.at[wid], ew_v)

    def chunk(j, carry):
        pltpu.async_copy(hp_hbm.at[row_v.at[j]], rows_v, sem).wait()
        jbase = j * CHUNK

        def group16(kk, kcarry):
            ew16 = ew_v[pl.ds(jbase + kk * 16, 16)]
            for t in range(16):
                g = jnp.broadcast_to(ew16[t], (16,))
                k16 = kk * 16 + t
                for f in range(F // 16):
                    sl = pl.ds(f * 16, 16)
                    rows_v[k16, sl] = rows_v[k16, sl] * g
            return kcarry
        lax.fori_loop(0, CHUNK // 16, group16, 0)
        pltpu.sync_copy(rows_v, acc_sh.at[col_v.at[j]], add=True)
        return carry
    lax.fori_loop(0, CH, chunk, 0)
    plsc.subcore_barrier()

    sl = pl.ds(s * SEG, SEG)
    pltpu.sync_copy(acc_sh.at[sl], out_hbm.at[c, sl])


_edge_kernel = functools.partial(
    pl.kernel,
    out_type=jax.ShapeDtypeStruct((NC, NROW, F), jnp.float32),
    mesh=_MESH,
    scratch_types=[
        pltpu.VMEM((CH, CHUNK), jnp.int32),
        pltpu.VMEM((CH, CHUNK), jnp.int32),
        pltpu.VMEM((PW,), jnp.float32),
        pltpu.VMEM((CHUNK, F), jnp.float32),
        pltpu.VMEM_SHARED((NROW, F), jnp.float32),
        pltpu.SemaphoreType.DMA,
    ],
)(_edge_body)


_BLK = 1000
_NBLK = N // _BLK


def _mm_scale_body(x_ref, w_ref, degT_ref, h_ref, dinv_ref):
    deg = degT_ref[:, 0:1] + degT_ref[:, 1:2]
    dinv = jnp.where(deg > 0, lax.rsqrt(jnp.maximum(deg, 1e-30)), 0.0)
    h = jnp.dot(x_ref[...], w_ref[...], precision=_HI,
                preferred_element_type=jnp.float32)
    h_ref[...] = h * dinv
    dinv_ref[...] = dinv


def _tc_mm_scale(x, w, degT):
    return pl.pallas_call(
        _mm_scale_body,
        grid=(_NBLK,),
        in_specs=[
            pl.BlockSpec((_BLK, F), lambda i: (i, 0)),
            pl.BlockSpec((F, F), lambda i: (0, 0)),
            pl.BlockSpec((_BLK, NC), lambda i: (i, 0)),
        ],
        out_specs=[
            pl.BlockSpec((_BLK, F), lambda i: (i, 0)),
            pl.BlockSpec((_BLK, 1), lambda i: (i, 0)),
        ],
        out_shape=[
            jax.ShapeDtypeStruct((N, F), jnp.float32),
            jax.ShapeDtypeStruct((N, 1), jnp.float32),
        ],
    )(x, w, degT)


def _stats_body(acc_ref, dinv_ref, y_ref, s1_ref, s2_ref):
    i = pl.program_id(0)
    y = (acc_ref[0] + acc_ref[1]) * dinv_ref[...]
    y_ref[...] = y
    s1 = jnp.sum(y, axis=0, keepdims=True)
    s2 = jnp.sum(y * y, axis=0, keepdims=True)

    @pl.when(i == 0)
    def _():
        s1_ref[...] = s1
        s2_ref[...] = s2

    @pl.when(i > 0)
    def _():
        s1_ref[...] += s1
        s2_ref[...] += s2


def _tc_stats(acc, dinv):
    return pl.pallas_call(
        _stats_body,
        grid=(_NBLK,),
        in_specs=[
            pl.BlockSpec((NC, _BLK, F), lambda i: (0, i, 0)),
            pl.BlockSpec((_BLK, 1), lambda i: (i, 0)),
        ],
        out_specs=[
            pl.BlockSpec((_BLK, F), lambda i: (i, 0)),
            pl.BlockSpec((1, F), lambda i: (0, 0)),
            pl.BlockSpec((1, F), lambda i: (0, 0)),
        ],
        out_shape=[
            jax.ShapeDtypeStruct((N, F), jnp.float32),
            jax.ShapeDtypeStruct((1, F), jnp.float32),
            jax.ShapeDtypeStruct((1, F), jnp.float32),
        ],
    )(acc, dinv)


def _bn_elu(y, s1_ref, s2_ref, g_ref, b_ref):
    mu = s1_ref[...] / N
    var = s2_ref[...] / N - mu * mu
    rstd = lax.rsqrt(var + EPS)
    z = (y - mu) * rstd * g_ref[...] + b_ref[...]
    return jnp.where(z > 0, z, jnp.exp(jnp.minimum(z, 0.0)) - 1.0)


def _bn_mm_body(y_ref, s1_ref, s2_ref, g_ref, b_ref, w_ref, dinv_ref, h_ref):
    z = _bn_elu(y_ref[...], s1_ref, s2_ref, g_ref, b_ref)
    h = jnp.dot(z, w_ref[...], precision=_HI,
                preferred_element_type=jnp.float32)
    h_ref[...] = h * dinv_ref[...]


def _tc_bn_mm(y, s1, s2, g, b, w, dinv):
    return pl.pallas_call(
        _bn_mm_body,
        grid=(_NBLK,),
        in_specs=[
            pl.BlockSpec((_BLK, F), lambda i: (i, 0)),
            pl.BlockSpec((1, F), lambda i: (0, 0)),
            pl.BlockSpec((1, F), lambda i: (0, 0)),
            pl.BlockSpec((1, F), lambda i: (0, 0)),
            pl.BlockSpec((1, F), lambda i: (0, 0)),
            pl.BlockSpec((F, F), lambda i: (0, 0)),
            pl.BlockSpec((_BLK, 1), lambda i: (i, 0)),
        ],
        out_specs=pl.BlockSpec((_BLK, F), lambda i: (i, 0)),
        out_shape=jax.ShapeDtypeStruct((N, F), jnp.float32),
    )(y, s1, s2, g, b, w, dinv)


def _pool_body(y_ref, s1_ref, s2_ref, g_ref, b_ref, batch_ref, lnw_ref,
               lnb_ref, out_ref, pooled_scr, cnt_scr):
    i = pl.program_id(0)
    z = _bn_elu(y_ref[...], s1_ref, s2_ref, g_ref, b_ref)
    gids = lax.broadcasted_iota(jnp.int32, (_BLK, G), 1)
    oh = (gids == jnp.broadcast_to(batch_ref[...], (_BLK, G))).astype(
        jnp.float32)

    @pl.when(i == 0)
    def _():
        pooled_scr[...] = jnp.zeros_like(pooled_scr)
        cnt_scr[...] = jnp.zeros_like(cnt_scr)

    pooled_scr[...] += lax.dot_general(
        oh, z, (((0,), (0,)), ((), ())),
        precision=_HI, preferred_element_type=jnp.float32)
    cnt_scr[...] += lax.dot_general(
        oh, jnp.ones((_BLK, 1), jnp.float32), (((0,), (0,)), ((), ())),
        precision=_HI, preferred_element_type=jnp.float32)

    @pl.when(i == _NBLK - 1)
    def _():
        pooled = pooled_scr[...] / jnp.maximum(cnt_scr[...], 1.0)
        out_ref[...] = lax.dot_general(
            pooled, lnw_ref[...], (((1,), (1,)), ((), ())),
            precision=_HI, preferred_element_type=jnp.float32) + lnb_ref[...]


def _tc_pool(y, s1, s2, g, b, batch_row, lnw, lnb):
    return pl.pallas_call(
        _pool_body,
        grid=(_NBLK,),
        in_specs=[
            pl.BlockSpec((_BLK, F), lambda i: (i, 0)),
            pl.BlockSpec((1, F), lambda i: (0, 0)),
            pl.BlockSpec((1, F), lambda i: (0, 0)),
            pl.BlockSpec((1, F), lambda i: (0, 0)),
            pl.BlockSpec((1, F), lambda i: (0, 0)),
            pl.BlockSpec((_BLK, 1), lambda i: (i, 0)),
            pl.BlockSpec((NCLS, F), lambda i: (0, 0)),
            pl.BlockSpec((1, NCLS), lambda i: (0, 0)),
        ],
        out_specs=pl.BlockSpec((G, NCLS), lambda i: (0, 0)),
        out_shape=jax.ShapeDtypeStruct((G, NCLS), jnp.float32),
        scratch_shapes=[
            pltpu.VMEM((G, F), jnp.float32),
            pltpu.VMEM((G, 1), jnp.float32),
        ],
    )(y, s1, s2, g, b, batch_row, lnw, lnb)


def kernel(x, edge_index, edge_weight, batch, W1, bn1_g, bn1_b,
           W2, bn2_g, bn2_b, lnW, lnb):
    e = edge_index.shape[1]
    row = edge_index[0].astype(jnp.int32)
    col = edge_index[1].astype(jnp.int32)
    ew = edge_weight.astype(jnp.float32)
    npad = EP - e
    pad_idx = (jnp.arange(npad, dtype=jnp.int32) * 7) % N
    row = jnp.concatenate([row, pad_idx]).reshape(NW, CH, CHUNK)
    col = jnp.concatenate([col, pad_idx]).reshape(NW, CH, CHUNK)
    ew = jnp.concatenate([ew, jnp.zeros((npad,), jnp.float32)]
                         ).reshape(NW, CH, CHUNK)

    ew_flat = ew.reshape(NW, PW)

    deg2 = _deg_kernel(col, ew).reshape(NC, DPAD)
    degT = deg2.T[:N]

    h1p, dinv = _tc_mm_scale(x, W1, degT)
    acc1 = _edge_kernel(h1p, row, col, ew_flat)
    y1, s1a, s2a = _tc_stats(acc1, dinv)
    g1 = bn1_g.reshape(1, F)
    b1 = bn1_b.reshape(1, F)
    h2p = _tc_bn_mm(y1, s1a, s2a, g1, b1, W2, dinv)
    acc2 = _edge_kernel(h2p, row, col, ew_flat)
    y2, s1b, s2b = _tc_stats(acc2, dinv)

    batch_col = batch.astype(jnp.int32).reshape(N, 1)
    return _tc_pool(y2, s1b, s2b, bn2_g.reshape(1, F), bn2_b.reshape(1, F),
                    batch_col, lnW, lnb.reshape(1, NCLS))


# speedup vs baseline: 22.5320x; 1.5173x over previous
import functools

import jax
import jax.numpy as jnp
from jax import lax
from jax.experimental import pallas as pl
from jax.experimental.pallas import tpu as pltpu
from jax.experimental.pallas import tpu_sc as plsc

N = 10000
F = 128
G = 64
NCLS = 4
EPS = 1e-5

NC = 2
NS = 16
NW = NC * NS
CHUNK = 128
CH = 80
HCH = 40
PW = CH * CHUNK
EP = NW * PW
DPAD = 10240
DSEG = DPAD // NS
NROW = 10112
SEG = NROW // NS

_HI = jax.lax.Precision.HIGHEST

_MESH = plsc.VectorSubcoreMesh(
    core_axis_name="c", subcore_axis_name="s", num_cores=NC, num_subcores=NS)


def _deg_body(col_hbm, ew_hbm, out_hbm, col_v, ew_v, zbuf, deg_sh):
    c = lax.axis_index("c")
    s = lax.axis_index("s")
    wid = s * NC + c
    zv = jnp.zeros((16,), jnp.float32)

    def zero_zbuf(i, carry):
        zbuf[pl.ds(i * 16, 16)] = zv
        return carry
    lax.fori_loop(0, DSEG // 16, zero_zbuf, 0, unroll=True)
    pltpu.sync_copy(zbuf.at[pl.ds(0, DSEG)], deg_sh.at[pl.ds(s * DSEG, DSEG)])
    plsc.subcore_barrier()

    pltpu.sync_copy(col_hbm.at[wid], col_v)
    pltpu.sync_copy(ew_hbm.at[wid], ew_v)

    def chunk(j, carry):
        pltpu.sync_copy(ew_v.at[j], deg_sh.at[col_v.at[j]], add=True)
        return carry
    lax.fori_loop(0, CH, chunk, 0)
    plsc.subcore_barrier()
    pltpu.sync_copy(deg_sh.at[pl.ds(s * DSEG, DSEG)],
                    out_hbm.at[pl.ds(c * DPAD + s * DSEG, DSEG)])


_deg_kernel = functools.partial(
    pl.kernel,
    out_type=jax.ShapeDtypeStruct((NC * DPAD,), jnp.float32),
    mesh=_MESH,
    scratch_types=[
        pltpu.VMEM((CH, CHUNK), jnp.int32),
        pltpu.VMEM((CH, CHUNK), jnp.float32),
        pltpu.VMEM((DSEG,), jnp.float32),
        pltpu.VMEM_SHARED((DPAD,), jnp.float32),
    ],
)(_deg_body)


def _edge_body(hp_hbm, row_hbm, col_hbm, ew_hbm, out_hbm,
               row_v, col_v, ew_v, rows_a, rows_b, acc_sh, sem_a, sem_b):
    c = lax.axis_index("c")
    s = lax.axis_index("s")
    wid = s * NC + c
    zv = jnp.zeros((16,), jnp.float32)

    def zero_rows(i, carry):
        for f in range(F // 16):
            rows_a[i, pl.ds(f * 16, 16)] = zv
        return carry
    lax.fori_loop(0, CHUNK, zero_rows, 0)

    def zero_acc(i, carry):
        pltpu.sync_copy(rows_a.at[pl.ds(0, CHUNK)],
                        acc_sh.at[pl.ds(s * SEG + i * CHUNK, CHUNK)])
        return carry
    lax.fori_loop(0, 4, zero_acc, 0)
    pltpu.sync_copy(rows_a.at[pl.ds(0, SEG - 4 * CHUNK)],
                    acc_sh.at[pl.ds(s * SEG + 4 * CHUNK, SEG - 4 * CHUNK)])
    plsc.subcore_barrier()

    def process(j, cur, sem_cur, nxt, sem_nxt):
        pltpu.make_async_copy(hp_hbm.at[row_v.at[j]], cur, sem_cur).wait()
        jn = (j + 1) % HCH
        pltpu.async_copy(hp_hbm.at[row_v.at[jn]], nxt, sem_nxt)
        jbase = j * CHUNK

        @plsc.parallel_loop(0, CHUNK, step=16, unroll=2)
        def _(k0):
            ew16 = ew_v[pl.ds(jbase + k0, 16)]
            for t in range(16):
                g = jnp.broadcast_to(ew16[t], (16,))
                for f in range(F // 16):
                    sl = pl.ds(f * 16, 16)
                    cur[k0 + t, sl] = cur[k0 + t, sl] * g

        pltpu.sync_copy(cur, acc_sh.at[col_v.at[j]], add=True)

    def pair(jj, carry):
        j = jj * 2
        process(j, rows_a, sem_a, rows_b, sem_b)
        process(j + 1, rows_b, sem_b, rows_a, sem_a)
        return carry

    for p in range(CH // HCH):
        pltpu.sync_copy(row_hbm.at[wid, pl.ds(p * HCH, HCH)], row_v)
        pltpu.sync_copy(col_hbm.at[wid, pl.ds(p * HCH, HCH)], col_v)
        pltpu.sync_copy(ew_hbm.at[wid, pl.ds(p * HCH * CHUNK, HCH * CHUNK)],
                        ew_v)
        pltpu.async_copy(hp_hbm.at[row_v.at[0]], rows_a, sem_a)
        lax.fori_loop(0, HCH // 2, pair, 0)
        pltpu.make_async_copy(hp_hbm.at[row_v.at[0]], rows_a, sem_a).wait()
    plsc.subcore_barrier()

    sl = pl.ds(s * SEG, SEG)
    pltpu.sync_copy(acc_sh.at[sl], out_hbm.at[c, sl])


_edge_kernel = functools.partial(
    pl.kernel,
    out_type=jax.ShapeDtypeStruct((NC, NROW, F), jnp.float32),
    mesh=_MESH,
    scratch_types=[
        pltpu.VMEM((HCH, CHUNK), jnp.int32),
        pltpu.VMEM((HCH, CHUNK), jnp.int32),
        pltpu.VMEM((HCH * CHUNK,), jnp.float32),
        pltpu.VMEM((CHUNK, F), jnp.float32),
        pltpu.VMEM((CHUNK, F), jnp.float32),
        pltpu.VMEM_SHARED((NROW, F), jnp.float32),
        pltpu.SemaphoreType.DMA,
        pltpu.SemaphoreType.DMA,
    ],
)(_edge_body)


def _mm_scale_body(x_ref, w_ref, degT_ref, h_ref, dinv_ref):
    deg = degT_ref[:, 0:1] + degT_ref[:, 1:2]
    dinv = jnp.where(deg > 0, lax.rsqrt(jnp.maximum(deg, 1e-30)), 0.0)
    h = jnp.dot(x_ref[...], w_ref[...], precision=_HI,
                preferred_element_type=jnp.float32)
    h_ref[...] = h * dinv
    dinv_ref[...] = dinv


def _tc_mm_scale(x, w, degT):
    return pl.pallas_call(
        _mm_scale_body,
        out_shape=[
            jax.ShapeDtypeStruct((N, F), jnp.float32),
            jax.ShapeDtypeStruct((N, 1), jnp.float32),
        ],
    )(x, w, degT)


def _combine_bn_elu(acc_ref, dinv_ref, g_ref, b_ref):
    dinv = dinv_ref[...]
    y = (acc_ref[0] + acc_ref[1]) * dinv
    mu = jnp.mean(y, axis=0, keepdims=True)
    var = jnp.mean(y * y, axis=0, keepdims=True) - mu * mu
    z = (y - mu) * lax.rsqrt(var + EPS) * g_ref[...] + b_ref[...]
    return jnp.where(z > 0, z, jnp.exp(jnp.minimum(z, 0.0)) - 1.0), dinv


def _bn_mm_body(acc_ref, dinv_ref, g_ref, b_ref, w_ref, h_ref):
    z, dinv = _combine_bn_elu(acc_ref, dinv_ref, g_ref, b_ref)
    h = jnp.dot(z, w_ref[...], precision=_HI,
                preferred_element_type=jnp.float32)
    h_ref[...] = h * dinv


def _tc_bn_mm(acc, dinv, g, b, w):
    return pl.pallas_call(
        _bn_mm_body,
        grid=(1,),
        in_specs=[
            pl.BlockSpec((NC, N, F), lambda i: (0, 0, 0)),
            pl.BlockSpec((N, 1), lambda i: (0, 0)),
            pl.BlockSpec((1, F), lambda i: (0, 0)),
            pl.BlockSpec((1, F), lambda i: (0, 0)),
            pl.BlockSpec((F, F), lambda i: (0, 0)),
        ],
        out_specs=pl.BlockSpec((N, F), lambda i: (0, 0)),
        out_shape=jax.ShapeDtypeStruct((N, F), jnp.float32),
    )(acc, dinv, g, b, w)


def _pool_body(acc_ref, dinv_ref, g_ref, b_ref, batch_ref, lnw_ref,
               lnb_ref, out_ref):
    z, _ = _combine_bn_elu(acc_ref, dinv_ref, g_ref, b_ref)
    gids = lax.broadcasted_iota(jnp.int32, (N, G), 1)
    oh = (gids == jnp.broadcast_to(batch_ref[...], (N, G))).astype(
        jnp.float32)
    pooled = lax.dot_general(
        oh, z, (((0,), (0,)), ((), ())),
        precision=_HI, preferred_element_type=jnp.float32)
    cnt = lax.dot_general(
        oh, jnp.ones((N, 1), jnp.float32), (((0,), (0,)), ((), ())),
        precision=_HI, preferred_element_type=jnp.float32)
    pooled = pooled / jnp.maximum(cnt, 1.0)
    out_ref[...] = lax.dot_general(
        pooled, lnw_ref[...], (((1,), (1,)), ((), ())),
        precision=_HI, preferred_element_type=jnp.float32) + lnb_ref[...]


def _tc_pool(acc, dinv, g, b, batch_col, lnw, lnb):
    return pl.pallas_call(
        _pool_body,
        grid=(1,),
        in_specs=[
            pl.BlockSpec((NC, N, F), lambda i: (0, 0, 0)),
            pl.BlockSpec((N, 1), lambda i: (0, 0)),
            pl.BlockSpec((1, F), lambda i: (0, 0)),
            pl.BlockSpec((1, F), lambda i: (0, 0)),
            pl.BlockSpec((N, 1), lambda i: (0, 0)),
            pl.BlockSpec((NCLS, F), lambda i: (0, 0)),
            pl.BlockSpec((1, NCLS), lambda i: (0, 0)),
        ],
        out_specs=pl.BlockSpec((G, NCLS), lambda i: (0, 0)),
        out_shape=jax.ShapeDtypeStruct((G, NCLS), jnp.float32),
    )(acc, dinv, g, b, batch_col, lnw, lnb)


def kernel(x, edge_index, edge_weight, batch, W1, bn1_g, bn1_b,
           W2, bn2_g, bn2_b, lnW, lnb):
    e = edge_index.shape[1]
    row = edge_index[0].astype(jnp.int32)
    col = edge_index[1].astype(jnp.int32)
    ew = edge_weight.astype(jnp.float32)
    npad = EP - e
    pad_idx = (jnp.arange(npad, dtype=jnp.int32) * 7) % N
    row = jnp.concatenate([row, pad_idx]).reshape(NW, CH, CHUNK)
    col = jnp.concatenate([col, pad_idx]).reshape(NW, CH, CHUNK)
    ew = jnp.concatenate([ew, jnp.zeros((npad,), jnp.float32)]
                         ).reshape(NW, CH, CHUNK)

    ew_flat = ew.reshape(NW, PW)

    deg2 = _deg_kernel(col, ew).reshape(NC, DPAD)
    degT = deg2.T[:N]

    h1p, dinv = _tc_mm_scale(x, W1, degT)
    acc1 = _edge_kernel(h1p, row, col, ew_flat)
    h2p = _tc_bn_mm(acc1, dinv, bn1_g.reshape(1, F), bn1_b.reshape(1, F), W2)
    acc2 = _edge_kernel(h2p, row, col, ew_flat)

    batch_col = batch.astype(jnp.int32).reshape(N, 1)
    return _tc_pool(acc2, dinv, bn2_g.reshape(1, F), bn2_b.reshape(1, F),
                    batch_col, lnW, lnb.reshape(1, NCLS))
